# Initial kernel scaffold; baseline (speedup 1.0000x reference)
import jax
import jax.numpy as jnp
from jax.experimental import pallas as pl


def kernel(x, edge_index, edge_weight):
    raise NotImplementedError("write your pallas kernel here")


# speedup vs baseline: 3.7519x; 3.7519x over previous
import functools

import jax
import jax.numpy as jnp
from jax import lax
from jax.experimental import pallas as pl
from jax.experimental.pallas import tpu as pltpu
from jax.experimental.pallas import tpu_sc as plsc

N_NODES = 10000
D_FEAT = 128
N_EDGES = 320000

NC = 2
NS = 16
LANES = 16
DH = D_FEAT // NC
CHUNK = 128
EDGES_PER_TILE = ((N_EDGES + NS * CHUNK - 1) // (NS * CHUNK)) * CHUNK
N_CHUNKS = EDGES_PER_TILE // CHUNK
E_PAD = EDGES_PER_TILE * NS
N_PAD = 10240
ROWS_PER_TILE = N_PAD // NS


def _sg_body(xs_hbm, src_hbm, dst_hbm, w_hbm, out_hbm,
             acc1, acc2, src_v, srcadj_v, dst_v, w_v, rows_v, sem):
    c = lax.axis_index("c")
    s = lax.axis_index("s")
    tile_base = s * EDGES_PER_TILE

    def zero_body(i, _):
        for j in range(DH // LANES):
            rows_v[i, pl.ds(j * LANES, LANES)] = jnp.zeros((LANES,),
                                                           jnp.float32)
        return 0
    lax.fori_loop(0, CHUNK, zero_body, 0)
    for k in range(ROWS_PER_TILE // CHUNK):
        row0 = s * ROWS_PER_TILE + k * CHUNK
        pltpu.sync_copy(rows_v, acc1.at[pl.ds(row0, CHUNK)])
        pltpu.sync_copy(rows_v, acc2.at[pl.ds(row0, CHUNK)])
    plsc.subcore_barrier()

    def edge_chunk(chunk, gather_hop1, acc_to):
        base = tile_base + chunk * CHUNK
        pltpu.sync_copy(src_hbm.at[pl.ds(base, CHUNK)], src_v)
        pltpu.sync_copy(dst_hbm.at[pl.ds(base, CHUNK)], dst_v)
        pltpu.sync_copy(w_hbm.at[pl.ds(base, CHUNK)], w_v)
        if gather_hop1:
            off = c * N_PAD
            for j in range(CHUNK // LANES):
                srcadj_v[pl.ds(j * LANES, LANES)] = (
                    src_v[pl.ds(j * LANES, LANES)] + off)
            pltpu.async_copy(xs_hbm.at[srcadj_v], rows_v, sem).wait()
        else:
            pltpu.async_copy(acc1.at[src_v], rows_v, sem).wait()
        for g in range(CHUNK // LANES):
            wv = w_v[pl.ds(g * LANES, LANES)]
            for l in range(LANES):
                w16 = jnp.broadcast_to(wv[l], (LANES,))
                for j in range(DH // LANES):
                    sl = pl.ds(j * LANES, LANES)
                    rows_v[g * LANES + l, sl] = rows_v[g * LANES + l,
                                                       sl] * w16
        pltpu.sync_copy(rows_v, acc_to.at[dst_v], add=True)

    def hop1_body(chunk, _):
        edge_chunk(chunk, True, acc1)
        return 0
    lax.fori_loop(0, N_CHUNKS, hop1_body, 0)
    plsc.subcore_barrier()

    def hop2_body(chunk, _):
        edge_chunk(chunk, False, acc2)
        return 0
    lax.fori_loop(0, N_CHUNKS, hop2_body, 0)
    plsc.subcore_barrier()

    for k in range(ROWS_PER_TILE // CHUNK):
        row0 = s * ROWS_PER_TILE + k * CHUNK
        pltpu.sync_copy(acc2.at[pl.ds(row0, CHUNK)],
                        out_hbm.at[pl.ds(c * N_PAD + row0, CHUNK)])


def _build_kernel():
    mesh = plsc.VectorSubcoreMesh(core_axis_name="c", subcore_axis_name="s")
    return pl.kernel(
        _sg_body,
        out_type=jax.ShapeDtypeStruct((NC * N_PAD, DH), jnp.float32),
        mesh=mesh,
        scratch_types=[
            pltpu.VMEM_SHARED((N_PAD, DH), jnp.float32),
            pltpu.VMEM_SHARED((N_PAD, DH), jnp.float32),
            pltpu.VMEM((CHUNK,), jnp.int32),
            pltpu.VMEM((CHUNK,), jnp.int32),
            pltpu.VMEM((CHUNK,), jnp.int32),
            pltpu.VMEM((CHUNK,), jnp.float32),
            pltpu.VMEM((CHUNK, DH), jnp.float32),
            pltpu.SemaphoreType.DMA,
        ],
        compiler_params=pltpu.CompilerParams(needs_layout_passes=False,
                                             use_tc_tiling_on_sc=False),
    )


@jax.jit
def _sg_conv(x, src, dst, w):
    pad = E_PAD - N_EDGES
    src = jnp.concatenate([src, jnp.zeros((pad,), jnp.int32)])
    dst = jnp.concatenate([dst, jnp.zeros((pad,), jnp.int32)])
    w = jnp.concatenate([w, jnp.zeros((pad,), jnp.float32)])
    xs = x.reshape(N_NODES, NC, DH).transpose(1, 0, 2)
    xs = jnp.pad(xs, ((0, 0), (0, N_PAD - N_NODES), (0, 0))).reshape(
        NC * N_PAD, DH)
    out = _build_kernel()(xs, src, dst, w)
    return out.reshape(NC, N_PAD, DH)[:, :N_NODES].transpose(
        1, 0, 2).reshape(N_NODES, D_FEAT)


def kernel(x, edge_index, edge_weight):
    src = edge_index[0].astype(jnp.int32)
    dst = edge_index[1].astype(jnp.int32)
    return _sg_conv(x.astype(jnp.float32), src, dst,
                    edge_weight.astype(jnp.float32))


# speedup vs baseline: 4.0874x; 1.0894x over previous
import jax
import jax.numpy as jnp
from jax import lax
from jax.experimental import pallas as pl
from jax.experimental.pallas import tpu as pltpu
from jax.experimental.pallas import tpu_sc as plsc

N_NODES = 10000
D_FEAT = 128
N_EDGES = 320000

NC = 2
NS = 16
LANES = 16
DH = D_FEAT // NC
CHUNK = 128
NBUF = 2
GCH = NBUF * CHUNK
N_CHUNKS = 160
NGROUPS = N_CHUNKS // NBUF
EDGES_PER_TILE = N_CHUNKS * CHUNK
E_PAD = EDGES_PER_TILE * NS
N_PAD = 10240
ROWS_PER_TILE = N_PAD // NS


def _sg_body(xs_hbm, src_hbm, dst_hbm, w_hbm, out_hbm,
             acc1, acc2,
             srcA, srcB, wA, wB, dstA, dstB,
             rA0, rA1, rB0, rB1,
             gA0, gA1, gB0, gB1,
             sA0, sA1, sB0, sB1,
             iA, iB):
    c = lax.axis_index("c")
    s = lax.axis_index("s")
    tile_base = s * EDGES_PER_TILE
    src_v = [srcA, srcB]
    w_v = [wA, wB]
    dst_v = [dstA, dstB]
    rows = [[rA0, rA1], [rB0, rB1]]
    gsem = [[gA0, gA1], [gB0, gB1]]
    ssem = [[sA0, sA1], [sB0, sB1]]
    isem = [iA, iB]
    hbm_dummy = xs_hbm.at[pl.ds(0, CHUNK)]

    def zero_body(i, _):
        for j in range(DH // LANES):
            rA0[i, pl.ds(j * LANES, LANES)] = jnp.zeros((LANES,),
                                                        jnp.float32)
        return 0
    lax.fori_loop(0, CHUNK, zero_body, 0)
    for k in range(ROWS_PER_TILE // CHUNK):
        row0 = s * ROWS_PER_TILE + k * CHUNK
        pltpu.sync_copy(rA0, acc1.at[pl.ds(row0, CHUNK)])
        pltpu.sync_copy(rA0, acc2.at[pl.ds(row0, CHUNK)])
    plsc.subcore_barrier()

    def fetch(grp, P, first_hop, acc_from, wait_scatter):
        base = tile_base + grp * GCH
        descs = [
            pltpu.async_copy(src_hbm.at[pl.ds(base, GCH)], src_v[P],
                             isem[P]),
            pltpu.async_copy(w_hbm.at[pl.ds(base, GCH)], w_v[P], isem[P]),
        ]
        for b in range(NBUF):
            descs.append(pltpu.async_copy(
                dst_hbm.at[pl.ds(base + b * CHUNK, CHUNK)],
                dst_v[P].at[b], isem[P]))
        for d in descs:
            d.wait()
        if first_hop:
            off = c * N_PAD
            for j in range(GCH // LANES):
                sl = pl.ds(j * LANES, LANES)
                src_v[P][sl] = src_v[P][sl] + off
        gsrc = xs_hbm if first_hop else acc_from
        for b in range(NBUF):
            if wait_scatter:
                pltpu.make_async_copy(hbm_dummy, rows[P][b],
                                      ssem[P][b]).wait()
            pltpu.async_copy(
                gsrc.at[src_v[P].at[pl.ds(b * CHUNK, CHUNK)]],
                rows[P][b], gsem[P][b])

    def process(P, acc_to):
        for b in range(NBUF):
            pltpu.make_async_copy(hbm_dummy, rows[P][b],
                                  gsem[P][b]).wait()
            rows_b = rows[P][b]

            def mul_body(g, _, P=P, b=b, rows_b=rows_b):
                wv = w_v[P][pl.ds(b * CHUNK + g * LANES, LANES)]
                for l in range(LANES):
                    w16 = jnp.broadcast_to(wv[l], (LANES,))
                    i = g * LANES + l
                    for j in range(DH // LANES):
                        sl = pl.ds(j * LANES, LANES)
                        rows_b[i, sl] = rows_b[i, sl] * w16
                return 0
            lax.fori_loop(0, CHUNK // LANES, mul_body, 0)
            pltpu.async_copy(rows[P][b], acc_to.at[dst_v[P].at[b]],
                             ssem[P][b], add=True)

    def run_hop(first_hop, acc_from, acc_to):
        fetch(0, 0, first_hop, acc_from, False)
        fetch(1, 1, first_hop, acc_from, False)

        def pair_body(g, _):
            process(0, acc_to)
            fetch(2 * g + 2, 0, first_hop, acc_from, True)
            process(1, acc_to)
            fetch(2 * g + 3, 1, first_hop, acc_from, True)
            return 0
        lax.fori_loop(0, NGROUPS // 2 - 1, pair_body, 0)
        process(0, acc_to)
        process(1, acc_to)
        for P in range(2):
            for b in range(NBUF):
                pltpu.make_async_copy(hbm_dummy, rows[P][b],
                                      ssem[P][b]).wait()

    run_hop(True, None, acc1)
    plsc.subcore_barrier()
    run_hop(False, acc1, acc2)
    plsc.subcore_barrier()

    for k in range(ROWS_PER_TILE // CHUNK):
        row0 = s * ROWS_PER_TILE + k * CHUNK
        pltpu.sync_copy(acc2.at[pl.ds(row0, CHUNK)],
                        out_hbm.at[pl.ds(c * N_PAD + row0, CHUNK)])


def _build_kernel():
    mesh = plsc.VectorSubcoreMesh(core_axis_name="c", subcore_axis_name="s")
    return pl.kernel(
        _sg_body,
        out_type=jax.ShapeDtypeStruct((NC * N_PAD, DH), jnp.float32),
        mesh=mesh,
        scratch_types=[
            pltpu.VMEM_SHARED((N_PAD, DH), jnp.float32),
            pltpu.VMEM_SHARED((N_PAD, DH), jnp.float32),
            pltpu.VMEM((GCH,), jnp.int32),
            pltpu.VMEM((GCH,), jnp.int32),
            pltpu.VMEM((GCH,), jnp.float32),
            pltpu.VMEM((GCH,), jnp.float32),
            pltpu.VMEM((NBUF, CHUNK), jnp.int32),
            pltpu.VMEM((NBUF, CHUNK), jnp.int32),
        ] + [pltpu.VMEM((CHUNK, DH), jnp.float32)] * 4
           + [pltpu.SemaphoreType.DMA] * 10,
        compiler_params=pltpu.CompilerParams(needs_layout_passes=False,
                                             use_tc_tiling_on_sc=False),
    )


@jax.jit
def _sg_conv(x, src, dst, w):
    pad = E_PAD - N_EDGES
    src = jnp.concatenate([src, jnp.zeros((pad,), jnp.int32)])
    dst = jnp.concatenate([dst, jnp.zeros((pad,), jnp.int32)])
    w = jnp.concatenate([w, jnp.zeros((pad,), jnp.float32)])
    xs = x.reshape(N_NODES, NC, DH).transpose(1, 0, 2)
    xs = jnp.pad(xs, ((0, 0), (0, N_PAD - N_NODES), (0, 0))).reshape(
        NC * N_PAD, DH)
    out = _build_kernel()(xs, src, dst, w)
    return out.reshape(NC, N_PAD, DH)[:, :N_NODES].transpose(
        1, 0, 2).reshape(N_NODES, D_FEAT)


def kernel(x, edge_index, edge_weight):
    src = edge_index[0].astype(jnp.int32)
    dst = edge_index[1].astype(jnp.int32)
    return _sg_conv(x.astype(jnp.float32), src, dst,
                    edge_weight.astype(jnp.float32))


# speedup vs baseline: 6.7421x; 1.6495x over previous
import jax
import jax.numpy as jnp
from jax import lax
from jax.experimental import pallas as pl
from jax.experimental.pallas import tpu as pltpu
from jax.experimental.pallas import tpu_sc as plsc

N_NODES = 10000
D_FEAT = 128
N_EDGES = 320000

NC = 2
NS = 16
LANES = 16
DH = D_FEAT // NC
CHUNK = 128
NBUF = 2
GCH = NBUF * CHUNK
N_CHUNKS = 160
NGROUPS = N_CHUNKS // NBUF
EDGES_PER_TILE = N_CHUNKS * CHUNK
E_PAD = EDGES_PER_TILE * NS
N_PAD = 10240
ROWS_PER_TILE = N_PAD // NS


def _sg_body(xs_hbm, src_hbm, dst_hbm, w_hbm, out_hbm,
             acc1, acc2,
             srcA, srcB, wA, wB, dstA, dstB,
             rA0, rA1, rB0, rB1,
             gA0, gA1, gB0, gB1,
             sA0, sA1, sB0, sB1,
             iA, iB):
    c = lax.axis_index("c")
    s = lax.axis_index("s")
    tile_base = s * EDGES_PER_TILE
    src_v = [srcA, srcB]
    w_v = [wA, wB]
    dst_v = [dstA, dstB]
    rows = [[rA0, rA1], [rB0, rB1]]
    gsem = [[gA0, gA1], [gB0, gB1]]
    ssem = [[sA0, sA1], [sB0, sB1]]
    isem = [iA, iB]
    hbm_dummy = xs_hbm.at[pl.ds(0, CHUNK)]

    def zero_body(i, _):
        for j in range(DH // LANES):
            rA0[i, pl.ds(j * LANES, LANES)] = jnp.zeros((LANES,),
                                                        jnp.float32)
        return 0
    lax.fori_loop(0, CHUNK, zero_body, 0)
    for k in range(ROWS_PER_TILE // CHUNK):
        row0 = s * ROWS_PER_TILE + k * CHUNK
        pltpu.sync_copy(rA0, acc1.at[pl.ds(row0, CHUNK)])
        pltpu.sync_copy(rA0, acc2.at[pl.ds(row0, CHUNK)])
    plsc.subcore_barrier()

    def fetch(grp, P, first_hop, acc_from, wait_scatter):
        base = tile_base + grp * GCH
        descs = [
            pltpu.async_copy(src_hbm.at[pl.ds(base, GCH)], src_v[P],
                             isem[P]),
            pltpu.async_copy(w_hbm.at[pl.ds(base, GCH)], w_v[P], isem[P]),
        ]
        for b in range(NBUF):
            descs.append(pltpu.async_copy(
                dst_hbm.at[pl.ds(base + b * CHUNK, CHUNK)],
                dst_v[P].at[b], isem[P]))
        for d in descs:
            d.wait()
        if first_hop:
            off = c * N_PAD
            for j in range(GCH // LANES):
                sl = pl.ds(j * LANES, LANES)
                src_v[P][sl] = src_v[P][sl] + off
        gsrc = xs_hbm if first_hop else acc_from
        for b in range(NBUF):
            if wait_scatter:
                pltpu.make_async_copy(hbm_dummy, rows[P][b],
                                      ssem[P][b]).wait()
            pltpu.async_copy(
                gsrc.at[src_v[P].at[pl.ds(b * CHUNK, CHUNK)]],
                rows[P][b], gsem[P][b])

    def process(P, acc_to):
        for b in range(NBUF):
            pltpu.make_async_copy(hbm_dummy, rows[P][b],
                                  gsem[P][b]).wait()
            rows_b = rows[P][b]

            @plsc.parallel_loop(0, CHUNK // LANES, 1, unroll=2)
            def mul_body(g, P=P, b=b, rows_b=rows_b):
                wv = w_v[P][pl.ds(b * CHUNK + g * LANES, LANES)]
                for l in range(LANES):
                    w16 = jnp.broadcast_to(wv[l], (LANES,))
                    i = g * LANES + l
                    for j in range(DH // LANES):
                        sl = pl.ds(j * LANES, LANES)
                        rows_b[i, sl] = rows_b[i, sl] * w16
            pltpu.async_copy(rows[P][b], acc_to.at[dst_v[P].at[b]],
                             ssem[P][b], add=True)

    def run_hop(first_hop, acc_from, acc_to):
        fetch(0, 0, first_hop, acc_from, False)
        fetch(1, 1, first_hop, acc_from, False)

        def pair_body(g, _):
            process(0, acc_to)
            fetch(2 * g + 2, 0, first_hop, acc_from, True)
            process(1, acc_to)
            fetch(2 * g + 3, 1, first_hop, acc_from, True)
            return 0
        lax.fori_loop(0, NGROUPS // 2 - 1, pair_body, 0)
        process(0, acc_to)
        process(1, acc_to)
        for P in range(2):
            for b in range(NBUF):
                pltpu.make_async_copy(hbm_dummy, rows[P][b],
                                      ssem[P][b]).wait()

    run_hop(True, None, acc1)
    plsc.subcore_barrier()
    run_hop(False, acc1, acc2)
    plsc.subcore_barrier()

    for k in range(ROWS_PER_TILE // CHUNK):
        row0 = s * ROWS_PER_TILE + k * CHUNK
        pltpu.sync_copy(acc2.at[pl.ds(row0, CHUNK)],
                        out_hbm.at[pl.ds(c * N_PAD + row0, CHUNK)])


def _build_kernel():
    mesh = plsc.VectorSubcoreMesh(core_axis_name="c", subcore_axis_name="s")
    return pl.kernel(
        _sg_body,
        out_type=jax.ShapeDtypeStruct((NC * N_PAD, DH), jnp.float32),
        mesh=mesh,
        scratch_types=[
            pltpu.VMEM_SHARED((N_PAD, DH), jnp.float32),
            pltpu.VMEM_SHARED((N_PAD, DH), jnp.float32),
            pltpu.VMEM((GCH,), jnp.int32),
            pltpu.VMEM((GCH,), jnp.int32),
            pltpu.VMEM((GCH,), jnp.float32),
            pltpu.VMEM((GCH,), jnp.float32),
            pltpu.VMEM((NBUF, CHUNK), jnp.int32),
            pltpu.VMEM((NBUF, CHUNK), jnp.int32),
        ] + [pltpu.VMEM((CHUNK, DH), jnp.float32)] * 4
           + [pltpu.SemaphoreType.DMA] * 10,
        compiler_params=pltpu.CompilerParams(needs_layout_passes=False,
                                             use_tc_tiling_on_sc=False),
    )


@jax.jit
def _sg_conv(x, src, dst, w):
    pad = E_PAD - N_EDGES
    src = jnp.concatenate([src, jnp.zeros((pad,), jnp.int32)])
    dst = jnp.concatenate([dst, jnp.zeros((pad,), jnp.int32)])
    w = jnp.concatenate([w, jnp.zeros((pad,), jnp.float32)])
    xs = x.reshape(N_NODES, NC, DH).transpose(1, 0, 2)
    xs = jnp.pad(xs, ((0, 0), (0, N_PAD - N_NODES), (0, 0))).reshape(
        NC * N_PAD, DH)
    out = _build_kernel()(xs, src, dst, w)
    return out.reshape(NC, N_PAD, DH)[:, :N_NODES].transpose(
        1, 0, 2).reshape(N_NODES, D_FEAT)


def kernel(x, edge_index, edge_weight):
    src = edge_index[0].astype(jnp.int32)
    dst = edge_index[1].astype(jnp.int32)
    return _sg_conv(x.astype(jnp.float32), src, dst,
                    edge_weight.astype(jnp.float32))
